# baseline (speedup 1.0000x reference)
import functools

import jax
import jax.numpy as jnp
from jax.experimental import pallas as pl

_N_SAMPLES = 8
_BATCH = 4096
_C = 1000
_BB = 256


def _sampler_kernel(dist_ref, gu_ref, out_ref):
    d = dist_ref[...]
    s = jnp.sum(d, axis=-1, keepdims=True)
    p = d / s
    logp = jnp.log(p + 1e-9)
    entropy = -jnp.sum(p * logp, axis=-1)
    iota = jax.lax.broadcasted_iota(jnp.int32, (_BB, _C), 1)
    for n in range(_N_SAMPLES):
        u = gu_ref[n]
        g = -jnp.log(-jnp.log(u))
        k = logp + g
        idx = jnp.argmax(k, axis=-1)
        oh = iota == idx[:, None]
        log_prob = jnp.sum(jnp.where(oh, logp, 0.0), axis=-1)
        out_ref[n, :] = log_prob + entropy


@jax.jit
def kernel(dist, gumbel_u):
    return pl.pallas_call(
        _sampler_kernel,
        grid=(_BATCH // _BB,),
        in_specs=[
            pl.BlockSpec((_BB, _C), lambda i: (i, 0)),
            pl.BlockSpec((_N_SAMPLES, _BB, _C), lambda i: (0, i, 0)),
        ],
        out_specs=pl.BlockSpec((_N_SAMPLES, _BB), lambda i: (0, i)),
        out_shape=jax.ShapeDtypeStruct((_N_SAMPLES, _BATCH), jnp.float32),
    )(dist, gumbel_u)


# speedup vs baseline: 1.0854x; 1.0854x over previous
import jax
import jax.numpy as jnp
from jax.experimental import pallas as pl
from jax.experimental.pallas import tpu as pltpu

_N_SAMPLES = 8
_BATCH = 4096
_C = 1000
_BB = 512
_NB = _BATCH // _BB


def _sampler_kernel(dist_ref, gu_ref, out_ref, logp_ref, ent_ref):
    n = pl.program_id(1)

    @pl.when(n == 0)
    def _():
        d = dist_ref[...]
        s = jnp.sum(d, axis=-1, keepdims=True)
        p = d / s
        lp = jnp.log(p + 1e-9)
        logp_ref[...] = lp
        ent_ref[...] = -jnp.sum(p * lp, axis=-1, keepdims=True)

    logp = logp_ref[...]
    u = gu_ref[0]
    g = -jnp.log(-jnp.log(u))
    k = logp + g
    idx = jnp.argmax(k, axis=-1)
    iota = jax.lax.broadcasted_iota(jnp.int32, (_BB, _C), 1)
    log_prob = jnp.sum(jnp.where(iota == idx[:, None], logp, 0.0), axis=-1)
    out_ref[0, 0, :] = log_prob + ent_ref[:, 0]


@jax.jit
def kernel(dist, gumbel_u):
    out3 = pl.pallas_call(
        _sampler_kernel,
        grid=(_NB, _N_SAMPLES),
        in_specs=[
            pl.BlockSpec((_BB, _C), lambda i, n: (i, 0)),
            pl.BlockSpec((1, _BB, _C), lambda i, n: (n, i, 0)),
        ],
        out_specs=pl.BlockSpec((1, 1, _BB), lambda i, n: (n, 0, i)),
        out_shape=jax.ShapeDtypeStruct((_N_SAMPLES, 1, _BATCH), jnp.float32),
        scratch_shapes=[
            pltpu.VMEM((_BB, _C), jnp.float32),
            pltpu.VMEM((_BB, 1), jnp.float32),
        ],
    )(dist, gumbel_u)
    return out3.reshape(_N_SAMPLES, _BATCH)


# speedup vs baseline: 2.6047x; 2.3998x over previous
import jax
import jax.numpy as jnp
from jax.experimental import pallas as pl
from jax.experimental.pallas import tpu as pltpu

_N_SAMPLES = 8
_BATCH = 4096
_C = 1000
_BB = 512
_NB = _BATCH // _BB


def _sampler_kernel(dist_ref, gu_ref, out_ref, logp_ref, ent_ref):
    n = pl.program_id(1)

    @pl.when(n == 0)
    def _():
        d = dist_ref[...]
        s = jnp.sum(d, axis=0, keepdims=True)
        p = d / s
        lp = jnp.log(p + 1e-9)
        logp_ref[...] = lp
        ent_ref[...] = -jnp.sum(p * lp, axis=0, keepdims=True)

    logp = logp_ref[...]
    u = gu_ref[0]
    g = -jnp.log(-jnp.log(u))
    k = logp + g
    idx = jnp.argmax(k, axis=0)
    iota = jax.lax.broadcasted_iota(jnp.int32, (_C, _BB), 0)
    log_prob = jnp.sum(jnp.where(iota == idx[None, :], logp, 0.0), axis=0)
    out_ref[0, 0, :] = log_prob + ent_ref[0, :]


@jax.jit
def kernel(dist, gumbel_u):
    dist_t = jnp.transpose(dist)
    gu_t = jnp.transpose(gumbel_u, (0, 2, 1))
    out3 = pl.pallas_call(
        _sampler_kernel,
        grid=(_NB, _N_SAMPLES),
        in_specs=[
            pl.BlockSpec((_C, _BB), lambda i, n: (0, i)),
            pl.BlockSpec((1, _C, _BB), lambda i, n: (n, 0, i)),
        ],
        out_specs=pl.BlockSpec((1, 1, _BB), lambda i, n: (n, 0, i)),
        out_shape=jax.ShapeDtypeStruct((_N_SAMPLES, 1, _BATCH), jnp.float32),
        scratch_shapes=[
            pltpu.VMEM((_C, _BB), jnp.float32),
            pltpu.VMEM((1, _BB), jnp.float32),
        ],
    )(dist_t, gu_t)
    return out3.reshape(_N_SAMPLES, _BATCH)


# speedup vs baseline: 2.9183x; 1.1204x over previous
import jax
import jax.numpy as jnp
from jax.experimental import pallas as pl
from jax.experimental.pallas import tpu as pltpu

_N_SAMPLES = 8
_BATCH = 4096
_C = 1000
_BB = 512
_NB = _BATCH // _BB


def _sampler_kernel(dist_ref, gu_ref, out_ref, logp_ref, ent_ref):
    n = pl.program_id(1)

    @pl.when(n == 0)
    def _():
        d = dist_ref[...]
        s = jnp.sum(d, axis=0, keepdims=True)
        p = d / s
        lp = jnp.log(p + 1e-9)
        logp_ref[...] = lp
        ent_ref[...] = -jnp.sum(p * lp, axis=0, keepdims=True)

    def _tile_k(t):
        u = gu_ref[0, 8 * t:8 * t + 8, :]
        lp = logp_ref[8 * t:8 * t + 8, :]
        return lp + -jnp.log(-jnp.log(u)), lp

    k_acc, lp_acc = _tile_k(0)
    t_acc = jnp.zeros((8, _BB), jnp.int32)
    for t in range(1, _C // 8):
        kt, lpt = _tile_k(t)
        better = kt > k_acc
        k_acc = jnp.maximum(kt, k_acc)
        lp_acc = jnp.where(better, lpt, lp_acc)
        t_acc = jnp.where(better, jnp.int32(t), t_acc)

    m = jnp.max(k_acc, axis=0, keepdims=True)
    sub = jax.lax.broadcasted_iota(jnp.int32, (8, _BB), 0)
    c_acc = t_acc * 8 + sub
    cbest = jnp.min(jnp.where(k_acc == m, c_acc, jnp.int32(1 << 30)),
                    axis=0, keepdims=True)
    log_prob = jnp.sum(jnp.where(c_acc == cbest, lp_acc, 0.0), axis=0)
    out_ref[0, 0, :] = log_prob + ent_ref[0, :]


@jax.jit
def kernel(dist, gumbel_u):
    dist_t = jnp.transpose(dist)
    gu_t = jnp.transpose(gumbel_u, (0, 2, 1))
    out3 = pl.pallas_call(
        _sampler_kernel,
        grid=(_NB, _N_SAMPLES),
        in_specs=[
            pl.BlockSpec((_C, _BB), lambda i, n: (0, i)),
            pl.BlockSpec((1, _C, _BB), lambda i, n: (n, 0, i)),
        ],
        out_specs=pl.BlockSpec((1, 1, _BB), lambda i, n: (n, 0, i)),
        out_shape=jax.ShapeDtypeStruct((_N_SAMPLES, 1, _BATCH), jnp.float32),
        scratch_shapes=[
            pltpu.VMEM((_C, _BB), jnp.float32),
            pltpu.VMEM((1, _BB), jnp.float32),
        ],
    )(dist_t, gu_t)
    return out3.reshape(_N_SAMPLES, _BATCH)


# speedup vs baseline: 4.1281x; 1.4146x over previous
import jax
import jax.numpy as jnp
from jax.experimental import pallas as pl
from jax.experimental.pallas import tpu as pltpu

_N_SAMPLES = 8
_BATCH = 4096
_C = 1000
_BB = 512
_NB = _BATCH // _BB


def _sampler_kernel(dist_ref, gu_ref, out_ref):
    d = dist_ref[...]
    s = jnp.sum(d, axis=0, keepdims=True)
    p = d / s
    logp = jnp.log(p + 1e-9)
    ent = -jnp.sum(p * logp, axis=0, keepdims=True)

    sub = jax.lax.broadcasted_iota(jnp.int32, (8, _BB), 0)
    for n in range(_N_SAMPLES):
        def _tile_k(t):
            u = gu_ref[n, 8 * t:8 * t + 8, :]
            lp = logp[8 * t:8 * t + 8, :]
            return lp + -jnp.log(-jnp.log(u)), lp

        k_acc, lp_acc = _tile_k(0)
        t_acc = jnp.zeros((8, _BB), jnp.int32)
        for t in range(1, _C // 8):
            kt, lpt = _tile_k(t)
            better = kt > k_acc
            k_acc = jnp.maximum(kt, k_acc)
            lp_acc = jnp.where(better, lpt, lp_acc)
            t_acc = jnp.where(better, jnp.int32(t), t_acc)

        m = jnp.max(k_acc, axis=0, keepdims=True)
        c_acc = t_acc * 8 + sub
        cbest = jnp.min(jnp.where(k_acc == m, c_acc, jnp.int32(1 << 30)),
                        axis=0, keepdims=True)
        log_prob = jnp.sum(jnp.where(c_acc == cbest, lp_acc, 0.0),
                           axis=0, keepdims=True)
        out_ref[n, :] = (log_prob + ent)[0, :]


@jax.jit
def kernel(dist, gumbel_u):
    dist_t = jnp.transpose(dist)
    gu_t = jnp.transpose(gumbel_u, (0, 2, 1))
    return pl.pallas_call(
        _sampler_kernel,
        grid=(_NB,),
        in_specs=[
            pl.BlockSpec((_C, _BB), lambda i: (0, i)),
            pl.BlockSpec((_N_SAMPLES, _C, _BB), lambda i: (0, 0, i)),
        ],
        out_specs=pl.BlockSpec((_N_SAMPLES, _BB), lambda i: (0, i)),
        out_shape=jax.ShapeDtypeStruct((_N_SAMPLES, _BATCH), jnp.float32),
    )(dist_t, gu_t)
